# baseline (speedup 1.0000x reference)
import functools
import jax
import jax.numpy as jnp
from jax import lax
from jax.experimental import pallas as pl
from jax.experimental.pallas import tpu as pltpu
from jax.experimental.pallas import tpu_sc as plsc

D_EMB = 64
SCALE = 8.0
N_SEQ = 16384
N_TOK = 50
TOK_PAD = 56
X_LANES = 128
LANE_PAD = 128
NUM_CORES = 2
NUM_SUBCORES = 16
NUM_WORKERS = NUM_CORES * NUM_SUBCORES
ROWS_PER_WORKER = N_SEQ // NUM_WORKERS
STEPS = ROWS_PER_WORKER
ROW_ELEMS = N_TOK * D_EMB
NG = 6
NS = 3


def _scale_into(gbuf, sbuf):
    def jrow(j, carry):
        for c in range(D_EMB // 16):
            sl = pl.ds(c * 16, 16)
            sbuf[0, j, sl] = gbuf[j, sl] * SCALE
        return carry

    lax.fori_loop(0, N_TOK, jrow, 0)


def _emb_body(x_hbm, table_hbm, out_hbm, idx_v, gbufs, sbufs, gsems, ssems):
    w = lax.axis_index("s") * NUM_CORES + lax.axis_index("c")
    row0 = w * ROWS_PER_WORKER
    pltpu.sync_copy(
        x_hbm.at[pl.ds(row0, ROWS_PER_WORKER), pl.ds(0, TOK_PAD)], idx_v
    )

    def start_gather(g, b):
        pltpu.async_copy(
            table_hbm.at[idx_v.at[g]], gbufs[b], gsems[b]
        )

    def wait_gather(b):
        pltpu.make_async_copy(
            table_hbm.at[idx_v.at[0]], gbufs[b], gsems[b]
        ).wait()

    def start_scatter(g, s):
        pltpu.async_copy(
            sbufs[s], out_hbm.at[pl.ds(row0 + g, 1), :, :], ssems[s]
        )

    def wait_scatter(s):
        pltpu.make_async_copy(
            sbufs[s], out_hbm.at[pl.ds(0, 1), :, :], ssems[s]
        ).wait()

    for b in range(NG):
        start_gather(b, b)

    def visit(g, b, s, first, last):
        if not first:
            wait_scatter(s)
        wait_gather(b)
        _scale_into(gbufs[b], sbufs[s])
        start_scatter(g, s)
        if not last:
            start_gather(g + NG, b)

    for g in range(NG):
        visit(g, g % NG, g % NS, first=(g < NS), last=False)

    n_rounds = (STEPS - NG - 8) // NG

    def round_body(r, carry):
        g0 = NG + r * NG
        for t in range(NG):
            g = g0 + t
            visit(g, t % NG, t % NS, first=False, last=False)
        return carry

    lax.fori_loop(0, n_rounds, round_body, 0)

    tail0 = NG + n_rounds * NG
    for g in range(tail0, STEPS):
        visit(g, g % NG, g % NS, first=False, last=(g + NG >= STEPS))

    for s in range(NS):
        wait_scatter(s)


def kernel(x, table):
    mesh = plsc.VectorSubcoreMesh(core_axis_name="c", subcore_axis_name="s")
    fn = functools.partial(
        pl.kernel,
        mesh=mesh,
        out_type=jax.ShapeDtypeStruct((N_SEQ, TOK_PAD, LANE_PAD), jnp.float32),
        scratch_types=[
            pltpu.VMEM((ROWS_PER_WORKER, TOK_PAD), jnp.int32),
            [pltpu.VMEM((TOK_PAD, D_EMB), jnp.float32) for _ in range(NG)],
            [pltpu.VMEM((1, TOK_PAD, LANE_PAD), jnp.float32) for _ in range(NS)],
            [pltpu.SemaphoreType.DMA for _ in range(NG)],
            [pltpu.SemaphoreType.DMA for _ in range(NS)],
        ],
        compiler_params=pltpu.CompilerParams(use_tc_tiling_on_sc=False),
    )(_emb_body)
    x_p = jnp.pad(x.astype(jnp.int32), ((0, 0), (0, X_LANES - N_TOK)))
    out_pad = fn(x_p, table)
    return out_pad[:, :N_TOK, :D_EMB]


# speedup vs baseline: 2.7156x; 2.7156x over previous
import functools
import jax
import jax.numpy as jnp
from jax import lax
from jax.experimental import pallas as pl
from jax.experimental.pallas import tpu as pltpu
from jax.experimental.pallas import tpu_sc as plsc

D_EMB = 64
SCALE = 8.0
N_SEQ = 16384
N_TOK = 50
TOK_PAD = 56
X_LANES = 128
LANE_PAD = 128
NUM_CORES = 2
NUM_SUBCORES = 16
NUM_WORKERS = NUM_CORES * NUM_SUBCORES
ROWS_PER_WORKER = N_SEQ // NUM_WORKERS
STEPS = ROWS_PER_WORKER
ROW_ELEMS = N_TOK * D_EMB
NG = 6
NS = 3


def _scale_into(gbuf, sbuf):
    def jrow(j, carry):
        for c in range(D_EMB // 16):
            sl = pl.ds(c * 16, 16)
            sbuf[0, j, sl] = gbuf[j, sl] * SCALE
        return carry

    lax.fori_loop(0, N_TOK, jrow, 0)


def _emb_body(x_hbm, table_hbm, out_hbm, idx_v, gbufs, sbufs, gsems, ssems):
    w = lax.axis_index("s") * NUM_CORES + lax.axis_index("c")
    row0 = w * ROWS_PER_WORKER
    pltpu.sync_copy(
        x_hbm.at[pl.ds(row0, ROWS_PER_WORKER), pl.ds(0, TOK_PAD)], idx_v
    )

    def start_gather(g, b):
        pltpu.async_copy(
            table_hbm.at[idx_v.at[g]], gbufs[b], gsems[b]
        )

    def wait_gather(b):
        pltpu.make_async_copy(
            table_hbm.at[idx_v.at[0]], gbufs[b], gsems[b]
        ).wait()

    def start_scatter(g, s):
        pltpu.async_copy(
            sbufs[s], out_hbm.at[pl.ds(row0 + g, 1), :, :], ssems[s]
        )

    def wait_scatter(s):
        pltpu.make_async_copy(
            sbufs[s], out_hbm.at[pl.ds(0, 1), :, :], ssems[s]
        ).wait()

    for b in range(NG):
        start_gather(b, b)

    def visit(g, b, s, first, last):
        if not first:
            wait_scatter(s)
        wait_gather(b)
        _scale_into(gbufs[b], sbufs[s])
        start_scatter(g, s)
        if not last:
            start_gather(g + NG, b)

    for g in range(NG):
        visit(g, g % NG, g % NS, first=(g < NS), last=False)

    n_rounds = (STEPS - NG - 8) // NG

    def round_body(r, carry):
        g0 = NG + r * NG
        for t in range(NG):
            g = g0 + t
            visit(g, t % NG, t % NS, first=False, last=False)
        return carry

    lax.fori_loop(0, n_rounds, round_body, 0)

    tail0 = NG + n_rounds * NG
    for g in range(tail0, STEPS):
        visit(g, g % NG, g % NS, first=False, last=(g + NG >= STEPS))

    for s in range(NS):
        wait_scatter(s)


def kernel(x, table):
    mesh = plsc.VectorSubcoreMesh(core_axis_name="c", subcore_axis_name="s")
    fn = functools.partial(
        pl.kernel,
        mesh=mesh,
        out_type=jax.ShapeDtypeStruct((N_SEQ, TOK_PAD, LANE_PAD), jnp.float32),
        scratch_types=[
            pltpu.VMEM((ROWS_PER_WORKER, TOK_PAD), jnp.int32),
            [pltpu.VMEM((TOK_PAD, D_EMB), jnp.float32) for _ in range(NG)],
            [pltpu.VMEM((1, TOK_PAD, LANE_PAD), jnp.float32) for _ in range(NS)],
            [pltpu.SemaphoreType.DMA for _ in range(NG)],
            [pltpu.SemaphoreType.DMA for _ in range(NS)],
        ],
        compiler_params=pltpu.CompilerParams(use_tc_tiling_on_sc=False),
    )(_emb_body)
    x_i = x.astype(jnp.int32)
    x_p = jnp.concatenate([x_i, x_i[:, : TOK_PAD - N_TOK]], axis=1)
    x_p = jnp.pad(x_p, ((0, 0), (0, X_LANES - TOK_PAD)))
    out_pad = fn(x_p, table)
    return out_pad[:, :N_TOK, :D_EMB]


# speedup vs baseline: 2.7462x; 1.0113x over previous
import functools
import jax
import jax.numpy as jnp
from jax import lax
from jax.experimental import pallas as pl
from jax.experimental.pallas import tpu as pltpu
from jax.experimental.pallas import tpu_sc as plsc

D_EMB = 64
SCALE = 8.0
N_SEQ = 16384
N_TOK = 50
TOK_PAD = 56
LANE_PAD = 128
NUM_CORES = 2
NUM_SUBCORES = 16
NUM_WORKERS = NUM_CORES * NUM_SUBCORES
ROWS_PER_WORKER = N_SEQ // NUM_WORKERS
STEPS = ROWS_PER_WORKER
ROW_ELEMS = N_TOK * D_EMB
NG = 6
NS = 3


def _scale_into(gbuf, sbuf):
    def jrow(j, carry):
        for c in range(D_EMB // 16):
            sbuf[pl.ds(j * D_EMB + c * 16, 16)] = gbuf[j, pl.ds(c * 16, 16)] * SCALE
        return carry

    lax.fori_loop(0, N_TOK, jrow, 0)


def _emb_body(x_hbm, table_hbm, out_hbm, idx_v, gbufs, sbufs, gsems, ssems):
    w = lax.axis_index("s") * NUM_CORES + lax.axis_index("c")
    row0 = w * ROWS_PER_WORKER
    pltpu.sync_copy(x_hbm.at[pl.ds(row0, ROWS_PER_WORKER)], idx_v)

    def start_gather(g, b):
        pltpu.async_copy(
            table_hbm.at[idx_v.at[g]], gbufs[b], gsems[b]
        )

    def wait_gather(b):
        pltpu.make_async_copy(
            table_hbm.at[idx_v.at[0]], gbufs[b], gsems[b]
        ).wait()

    def start_scatter(g, s):
        pltpu.async_copy(
            sbufs[s], out_hbm.at[pl.ds((row0 + g) * ROW_ELEMS, ROW_ELEMS)], ssems[s]
        )

    def wait_scatter(s):
        pltpu.make_async_copy(
            sbufs[s], out_hbm.at[pl.ds(0, ROW_ELEMS)], ssems[s]
        ).wait()

    for b in range(NG):
        start_gather(b, b)

    def visit(g, b, s, first, last):
        if not first:
            wait_scatter(s)
        wait_gather(b)
        _scale_into(gbufs[b], sbufs[s])
        start_scatter(g, s)
        if not last:
            start_gather(g + NG, b)

    for g in range(NG):
        visit(g, g % NG, g % NS, first=(g < NS), last=False)

    n_rounds = (STEPS - NG - 8) // NG

    def round_body(r, carry):
        g0 = NG + r * NG
        for t in range(NG):
            g = g0 + t
            visit(g, t % NG, t % NS, first=False, last=False)
        return carry

    lax.fori_loop(0, n_rounds, round_body, 0)

    tail0 = NG + n_rounds * NG
    for g in range(tail0, STEPS):
        visit(g, g % NG, g % NS, first=False, last=(g + NG >= STEPS))

    for s in range(NS):
        wait_scatter(s)


def kernel(x, table):
    mesh = plsc.VectorSubcoreMesh(core_axis_name="c", subcore_axis_name="s")
    fn = functools.partial(
        pl.kernel,
        mesh=mesh,
        out_type=jax.ShapeDtypeStruct((N_SEQ * ROW_ELEMS,), jnp.float32),
        scratch_types=[
            pltpu.VMEM((ROWS_PER_WORKER, N_TOK), jnp.int32),
            [pltpu.VMEM((N_TOK, D_EMB), jnp.float32) for _ in range(NG)],
            [pltpu.VMEM((ROW_ELEMS,), jnp.float32) for _ in range(NS)],
            [pltpu.SemaphoreType.DMA for _ in range(NG)],
            [pltpu.SemaphoreType.DMA for _ in range(NS)],
        ],
        compiler_params=pltpu.CompilerParams(use_tc_tiling_on_sc=False),
    )(_emb_body)
    out_flat = fn(x.astype(jnp.int32), table)
    return out_flat.reshape(N_SEQ, N_TOK, D_EMB)
